# baseline (speedup 1.0000x reference)
import math

import jax
import jax.numpy as jnp
import numpy as np
from jax import lax
from jax.experimental import pallas as pl
from jax.experimental.pallas import tpu as pltpu
from jax.experimental.pallas import tpu_sc as plsc

V = 100000
D = 64
B = 4096
S = 200

NC = 2
NS = 16
NW = NC * NS
LB = B // NW
NBG = 4
NBO = 4
NGROUP = S // NBG
SCALE = math.sqrt(D)


def _positional_encoding():
    position = np.arange(0, S, dtype=np.float32)[:, None]
    div_term = np.exp(
        np.arange(0, D, 2, dtype=np.float32) * -(math.log(10000.0) / D))
    enc = np.zeros((S, D), dtype=np.float32)
    enc[:, 0::2] = np.sin(position * div_term)
    enc[:, 1::2] = np.cos(position * div_term)
    return enc


_ENC = _positional_encoding()

_mesh = plsc.VectorSubcoreMesh(
    core_axis_name="c", subcore_axis_name="s", num_cores=NC, num_subcores=NS)


@jax.jit
def _emb_kernel(xt, table, enc):
    @pl.kernel(
        out_type=jax.ShapeDtypeStruct((S, D // 8, NW, 8, LB), jnp.float32),
        mesh=_mesh,
        scratch_types=[
            pltpu.VMEM((S, LB), jnp.int32),
            pltpu.VMEM((S, D), jnp.float32),
            pltpu.VMEM((NBG, LB, D), jnp.float32),
            pltpu.VMEM((NBO, D // 8, 8, LB), jnp.float32),
            pltpu.VMEM(((LB // 16) * (D // 16), 16 * 17), jnp.float32),
            pltpu.SemaphoreType.DMA((NBG,)),
            pltpu.SemaphoreType.DMA((NBO,)),
        ],
        compiler_params=pltpu.CompilerParams(
            use_tc_tiling_on_sc=False, needs_layout_passes=False),
    )
    def body(xt_hbm, table_hbm, enc_hbm, out_hbm,
             idx_v, enc_v, gbuf, obuf, shear, gsem, osem):
        wid = lax.axis_index("s") * NC + lax.axis_index("c")
        pltpu.sync_copy(xt_hbm.at[:, pl.ds(wid * LB, LB)], idx_v)
        pltpu.sync_copy(enc_hbm, enc_v)

        lanes17 = lax.iota(jnp.int32, 16) * 17

        PROBE_NO_GATHER = False
        if not PROBE_NO_GATHER:
            for b in range(NBG):
                pltpu.async_copy(
                    table_hbm.at[idx_v.at[b]], gbuf.at[b], gsem.at[b])

        def group_body(g, carry):
            for b in range(NBG):
                s = g * NBG + b
                o = b % NBO
                if not PROBE_NO_GATHER:
                    pltpu.make_async_copy(
                        table_hbm.at[idx_v.at[s]], gbuf.at[b],
                        gsem.at[b]).wait()

                if not PROBE_NO_GATHER:
                    if b < NBO:
                        @pl.when(g > 0)
                        def _():
                            pltpu.make_async_copy(
                                obuf.at[o], out_hbm.at[0, :, 0],
                                osem.at[o]).wait()
                    else:
                        pltpu.make_async_copy(
                            obuf.at[o], out_hbm.at[0, :, 0], osem.at[o]).wait()

                @plsc.parallel_loop(0, (LB // 16) * (D // 16), unroll=2)
                def blk_body(i):
                    lb_i = i // (D // 16)
                    dc = i % (D // 16)
                    r0 = lb_i * 16
                    d0 = dc * 16
                    ev16 = enc_v[s, pl.ds(d0, 16)]
                    sc = shear.at[i]
                    for l in range(16):
                        y = gbuf[b, r0 + l, pl.ds(d0, 16)]
                        sc[pl.ds(l * 17, 16)] = y * SCALE + ev16
                    for k in range(16):
                        d = d0 + k
                        ov = plsc.load_gather(sc, [lanes17 + k])
                        obuf[o, d // 8, d % 8, pl.ds(r0, 16)] = ov

                sn = s + NBG

                if not PROBE_NO_GATHER:
                    @pl.when(sn < S)
                    def _():
                        pltpu.async_copy(
                            table_hbm.at[idx_v.at[sn]], gbuf.at[b],
                            gsem.at[b])

                if not PROBE_NO_GATHER:
                    pltpu.async_copy(
                        obuf.at[o], out_hbm.at[s, :, wid], osem.at[o])
            return carry

        lax.fori_loop(0, NGROUP, group_body, 0)

        if not PROBE_NO_GATHER:
            for o in range(NBO):
                pltpu.make_async_copy(
                    obuf.at[o], out_hbm.at[0, :, 0], osem.at[o]).wait()

    return body(xt, table, enc)


def kernel(x, table):
    p5 = _emb_kernel(x.T, table, _ENC)
    return jnp.transpose(p5, (2, 4, 0, 1, 3)).reshape(B, S, D)


# speedup vs baseline: 1.1047x; 1.1047x over previous
import math

import jax
import jax.numpy as jnp
import numpy as np
from jax import lax
from jax.experimental import pallas as pl
from jax.experimental.pallas import tpu as pltpu
from jax.experimental.pallas import tpu_sc as plsc

V = 100000
D = 64
B = 4096
S = 200

NC = 2
NS = 16
NW = NC * NS
LB = B // NW
NBG = 4
NBO = 4
NGROUP = S // NBG
SCALE = math.sqrt(D)


def _positional_encoding():
    position = np.arange(0, S, dtype=np.float32)[:, None]
    div_term = np.exp(
        np.arange(0, D, 2, dtype=np.float32) * -(math.log(10000.0) / D))
    enc = np.zeros((S, D), dtype=np.float32)
    enc[:, 0::2] = np.sin(position * div_term)
    enc[:, 1::2] = np.cos(position * div_term)
    return enc


_ENC = _positional_encoding()

_mesh = plsc.VectorSubcoreMesh(
    core_axis_name="c", subcore_axis_name="s", num_cores=NC, num_subcores=NS)


@jax.jit
def _emb_kernel(xt, table, enc):
    @pl.kernel(
        out_type=jax.ShapeDtypeStruct((S, D // 8, NW, 8, LB), jnp.float32),
        mesh=_mesh,
        scratch_types=[
            pltpu.VMEM((S // 8, 8, LB), jnp.int32),
            pltpu.VMEM((S, D), jnp.float32),
            pltpu.VMEM((NBG, LB, D), jnp.float32),
            pltpu.VMEM((NBO, D // 8, 8, LB), jnp.float32),
            pltpu.VMEM(((LB // 16) * (D // 16), 16 * 17), jnp.float32),
            pltpu.SemaphoreType.DMA((NBG,)),
            pltpu.SemaphoreType.DMA((NBO,)),
        ],
        compiler_params=pltpu.CompilerParams(
            use_tc_tiling_on_sc=False, needs_layout_passes=False),
    )
    def body(xt_hbm, table_hbm, enc_hbm, out_hbm,
             idx_v, enc_v, gbuf, obuf, shear, gsem, osem):
        wid = lax.axis_index("s") * NC + lax.axis_index("c")
        pltpu.sync_copy(xt_hbm.at[:, wid], idx_v)
        pltpu.sync_copy(enc_hbm, enc_v)

        lanes17 = lax.iota(jnp.int32, 16) * 17

        PROBE_NO_GATHER = False
        if not PROBE_NO_GATHER:
            for b in range(NBG):
                pltpu.async_copy(
                    table_hbm.at[idx_v.at[0, b]], gbuf.at[b], gsem.at[b])

        def group_body(g, carry):
            for b in range(NBG):
                s = g * NBG + b
                o = b % NBO
                if not PROBE_NO_GATHER:
                    pltpu.make_async_copy(
                        table_hbm.at[idx_v.at[s // 8, s % 8]], gbuf.at[b],
                        gsem.at[b]).wait()

                if not PROBE_NO_GATHER:
                    if b < NBO:
                        @pl.when(g > 0)
                        def _():
                            pltpu.make_async_copy(
                                obuf.at[o], out_hbm.at[0, :, 0],
                                osem.at[o]).wait()
                    else:
                        pltpu.make_async_copy(
                            obuf.at[o], out_hbm.at[0, :, 0], osem.at[o]).wait()

                @plsc.parallel_loop(0, (LB // 16) * (D // 16))
                def blk_body(i):
                    lb_i = i // (D // 16)
                    dc = i % (D // 16)
                    r0 = lb_i * 16
                    d0 = dc * 16
                    ev16 = enc_v[s, pl.ds(d0, 16)]
                    sc = shear.at[i]
                    for l in range(16):
                        y = gbuf[b, r0 + l, pl.ds(d0, 16)]
                        sc[pl.ds(l * 17, 16)] = y * SCALE + ev16
                    for k in range(16):
                        d = d0 + k
                        ov = plsc.load_gather(sc, [lanes17 + k])
                        obuf[o, d // 8, d % 8, pl.ds(r0, 16)] = ov

                sn = s + NBG

                if not PROBE_NO_GATHER:
                    @pl.when(sn < S)
                    def _():
                        pltpu.async_copy(
                            table_hbm.at[idx_v.at[sn // 8, sn % 8]],
                            gbuf.at[b], gsem.at[b])

                if not PROBE_NO_GATHER:
                    pltpu.async_copy(
                        obuf.at[o], out_hbm.at[s, :, wid], osem.at[o])
            return carry

        lax.fori_loop(0, NGROUP, group_body, 0)

        if not PROBE_NO_GATHER:
            for o in range(NBO):
                pltpu.make_async_copy(
                    obuf.at[o], out_hbm.at[0, :, 0], osem.at[o]).wait()

    return body(xt, table, enc)


def kernel(x, table):
    xt4 = x.T.reshape(S // 8, 8, NW, LB).transpose(0, 2, 1, 3)
    p5 = _emb_kernel(xt4, table, _ENC)
    return jnp.transpose(p5, (2, 4, 0, 1, 3)).reshape(B, S, D)
